# baseline (speedup 1.0000x reference)
import functools

import jax
import jax.numpy as jnp
from jax import lax
from jax.experimental import pallas as pl
from jax.experimental.pallas import tpu as pltpu
from jax.experimental.pallas import tpu_sc as plsc


def kernel(sequence, positions):
    B, S, D = sequence.shape
    _, N = positions.shape
    total = B * N

    seq2d = sequence.reshape(B * S, D)
    flat_idx = (
        positions.astype(jnp.int32)
        + (jnp.arange(B, dtype=jnp.int32) * S)[:, None]
    ).reshape(total)

    NC, NS = 2, 16
    NW = NC * NS
    rows = 40
    n_active = total // rows

    mesh = plsc.VectorSubcoreMesh(core_axis_name="c", subcore_axis_name="s")

    @functools.partial(
        pl.kernel,
        mesh=mesh,
        out_type=jax.ShapeDtypeStruct((total, D), jnp.float32),
        scratch_types=[
            pltpu.VMEM((rows,), jnp.int32),
            pltpu.VMEM((rows, D), jnp.float32),
            pltpu.SemaphoreType.DMA,
        ],
    )
    def gather_kernel(table_hbm, idx_hbm, out_hbm, idx_v, rows_v, sem):
        wid = lax.axis_index("s") * NC + lax.axis_index("c")

        @pl.when(wid < n_active)
        def _():
            base = wid * rows
            pltpu.sync_copy(idx_hbm.at[pl.ds(base, rows)], idx_v)
            pltpu.async_copy(table_hbm.at[idx_v], rows_v, sem).wait()
            pltpu.sync_copy(rows_v, out_hbm.at[pl.ds(base, rows)])

    out = gather_kernel(seq2d, flat_idx)
    return out.reshape(B, N, D)
